# baseline (speedup 1.0000x reference)
import jax
import jax.numpy as jnp
from jax import lax
from jax.experimental import pallas as pl
from jax.experimental.pallas import tpu as pltpu
from jax.experimental.pallas import tpu_sc as plsc

_N = 10000
_E = 320000
_H = 128
_NH = 4
_DH = _H // _NH
_TEMP = _H ** (-0.5)

_NC = 2
_NS = 16
_EPS = _E // _NS
_CH = 128
_NFULL = _EPS // _CH
_TAIL = _EPS - _NFULL * _CH
_NCHK = _NFULL + 1
_RPS = 640


def _sel_hd():
    d = lax.broadcasted_iota(jnp.int32, (_H, _NH), 0) // _DH
    h = lax.broadcasted_iota(jnp.int32, (_H, _NH), 1)
    return (d == h).astype(jnp.float32)


def _sel_dh():
    d = lax.broadcasted_iota(jnp.int32, (_NH, _H), 1) // _DH
    h = lax.broadcasted_iota(jnp.int32, (_NH, _H), 0)
    return (d == h).astype(jnp.float32)


def _edge_body(k_ref, q_ref, v_ref, ev_ref, exb_ref):
    kq = k_ref[...] * q_ref[...]
    s = lax.dot_general(kq, _sel_hd(), (((1,), (0,)), ((), ())),
                        preferred_element_type=jnp.float32) * _TEMP
    s = jnp.where(s >= 0, s, 0.2 * s)
    ex = jnp.exp(s)
    exb = lax.dot_general(ex, _sel_dh(), (((1,), (0,)), ((), ())),
                          preferred_element_type=jnp.float32)
    ev_ref[...] = v_ref[...] * exb
    exb_ref[...] = exb


def _combine_body(pn_ref, pd_ref, o_ref):
    num = pn_ref[...]
    den = pd_ref[...]
    o_ref[...] = jnp.where(den > 0, num / den, 0.0)


def _scatter_body(src_hbm, dst_hbm, z128, acc_out,
                  idx_v, evbuf, acc, sem):
    s = lax.axis_index("s")

    @pl.when(s < _NS - 1)
    def _():
        pltpu.sync_copy(z128.at[pl.ds(s * _RPS, _RPS)],
                        acc.at[pl.ds(s * _RPS, _RPS)])

    @pl.when(s == _NS - 1)
    def _():
        pltpu.sync_copy(z128.at[pl.ds(s * _RPS, _N - (_NS - 1) * _RPS)],
                        acc.at[pl.ds(s * _RPS, _N - (_NS - 1) * _RPS)])

    plsc.subcore_barrier()

    base = s * _EPS

    def run(src_hbm):
        def load(j, buf):
            pltpu.async_copy(
                src_hbm.at[pl.ds(base + j * _CH, _CH)],
                evbuf.at[buf], sem)
            pltpu.async_copy(dst_hbm.at[s].at[pl.ds(j * _CH, _CH)],
                             idx_v.at[buf], sem)

        def wait(j, buf):
            pltpu.make_async_copy(
                src_hbm.at[pl.ds(base + j * _CH, _CH)],
                evbuf.at[buf], sem).wait()
            pltpu.make_async_copy(dst_hbm.at[s].at[pl.ds(j * _CH, _CH)],
                                  idx_v.at[buf], sem).wait()

        load(0, 0)

        def chunk(j, carry):
            buf = lax.rem(j, 2)
            wait(j, buf)

            @pl.when(j + 1 < _NFULL)
            def _():
                load(j + 1, 1 - buf)

            pltpu.sync_copy(evbuf.at[buf], acc.at[idx_v.at[buf]], add=True)
            return carry

        lax.fori_loop(0, _NFULL, chunk, 0)
        pltpu.sync_copy(z128.at[pl.ds(0, _CH - _TAIL)],
                        evbuf.at[0].at[pl.ds(_TAIL, _CH - _TAIL)])
        pltpu.sync_copy(src_hbm.at[pl.ds(base + _NFULL * _CH, _TAIL)],
                        evbuf.at[0].at[pl.ds(0, _TAIL)])
        pltpu.sync_copy(dst_hbm.at[s].at[pl.ds(_NFULL * _CH, _CH)], idx_v.at[0])
        pltpu.sync_copy(evbuf.at[0], acc.at[idx_v.at[0]], add=True)

    run(src_hbm)

    plsc.subcore_barrier()

    @pl.when(s < _NS - 1)
    def _():
        pltpu.sync_copy(acc.at[pl.ds(s * _RPS, _RPS)],
                        acc_out.at[pl.ds(s * _RPS, _RPS)])

    @pl.when(s == _NS - 1)
    def _():
        pltpu.sync_copy(acc.at[pl.ds(s * _RPS, _N - (_NS - 1) * _RPS)],
                        acc_out.at[pl.ds(s * _RPS, _N - (_NS - 1) * _RPS)])


def kernel(keys, queries, values, edge_index):
    f32 = jnp.float32
    be = 2000
    ev, exb = pl.pallas_call(
        _edge_body,
        grid=(_E // be,),
        in_specs=[pl.BlockSpec((be, _H), lambda i: (i, 0))] * 3,
        out_specs=[pl.BlockSpec((be, _H), lambda i: (i, 0)),
                   pl.BlockSpec((be, _H), lambda i: (i, 0))],
        out_shape=[jax.ShapeDtypeStruct((_E, _H), f32),
                   jax.ShapeDtypeStruct((_E, _H), f32)],
    )(keys, queries, values)

    pad = _NCHK * _CH - _EPS
    dst3 = jnp.pad(edge_index[1].reshape(_NS, _EPS), ((0, 0), (0, pad)))
    dst3 = dst3.reshape(_NS, _NCHK * _CH)
    z128 = jnp.zeros((_N, _H), f32)

    mesh = plsc.VectorSubcoreMesh(core_axis_name="c", subcore_axis_name="s",
                                  num_cores=1)

    def scatter(src):
        return pl.kernel(
            _scatter_body,
            out_type=jax.ShapeDtypeStruct((_N, _H), f32),
            mesh=mesh,
            scratch_types=[
                pltpu.VMEM((2, _CH), jnp.int32),
                pltpu.VMEM((2, _CH, _H), f32),
                pltpu.VMEM_SHARED((_N, _H), f32),
                pltpu.SemaphoreType.DMA,
            ],
        )(src, dst3, z128)

    pnum = scatter(ev)
    pden = scatter(exb)

    bn = 2000
    out = pl.pallas_call(
        _combine_body,
        grid=(_N // bn,),
        in_specs=[pl.BlockSpec((bn, _H), lambda i: (i, 0)),
                  pl.BlockSpec((bn, _H), lambda i: (i, 0))],
        out_specs=pl.BlockSpec((bn, _H), lambda i: (i, 0)),
        out_shape=jax.ShapeDtypeStruct((_N, _H), f32),
    )(pnum, pden)
    return out


# speedup vs baseline: 1.5692x; 1.5692x over previous
import jax
import jax.numpy as jnp
from jax import lax
from jax.experimental import pallas as pl
from jax.experimental.pallas import tpu as pltpu
from jax.experimental.pallas import tpu_sc as plsc

_N = 10000
_E = 320000
_H = 128
_NH = 4
_DH = _H // _NH
_TEMP = _H ** (-0.5)

_NC = 2
_NS = 16
_EPS = _E // _NS
_CH = 128
_NFULL = _EPS // _CH
_TAIL = _EPS - _NFULL * _CH
_NCHK = _NFULL + 1
_NBUF = 3
_RPS = 640


def _sel_hd():
    d = lax.broadcasted_iota(jnp.int32, (_H, _NH), 0) // _DH
    h = lax.broadcasted_iota(jnp.int32, (_H, _NH), 1)
    return (d == h).astype(jnp.float32)


def _sel_dh():
    d = lax.broadcasted_iota(jnp.int32, (_NH, _H), 1) // _DH
    h = lax.broadcasted_iota(jnp.int32, (_NH, _H), 0)
    return (d == h).astype(jnp.float32)


def _edge_body(k_ref, q_ref, v_ref, ev_ref, exb_ref):
    kq = k_ref[...] * q_ref[...]
    s = lax.dot_general(kq, _sel_hd(), (((1,), (0,)), ((), ())),
                        preferred_element_type=jnp.float32) * _TEMP
    s = jnp.where(s >= 0, s, 0.2 * s)
    ex = jnp.exp(s)
    exb = lax.dot_general(ex, _sel_dh(), (((1,), (0,)), ((), ())),
                          preferred_element_type=jnp.float32)
    ev_ref[...] = v_ref[...] * exb
    exb_ref[...] = exb


def _combine_body(pn_ref, pd_ref, o_ref):
    num = pn_ref[0]
    den = pd_ref[0]
    o_ref[...] = jnp.where(den > 0, num / den, 0.0)


def _scatter_body(ev_hbm, exb_hbm, dst_hbm, z128, acc_out,
                  idx_v, evbuf, acc, sem, sem2):
    c = lax.axis_index("c")
    s = lax.axis_index("s")

    @pl.when(s < _NS - 1)
    def _():
        pltpu.sync_copy(z128.at[pl.ds(s * _RPS, _RPS)],
                        acc.at[pl.ds(s * _RPS, _RPS)])

    @pl.when(s == _NS - 1)
    def _():
        pltpu.sync_copy(z128.at[pl.ds(s * _RPS, _N - (_NS - 1) * _RPS)],
                        acc.at[pl.ds(s * _RPS, _N - (_NS - 1) * _RPS)])

    plsc.subcore_barrier()

    base = s * _EPS

    def run(src_hbm):
        def load(j, buf):
            pltpu.async_copy(
                src_hbm.at[pl.ds(base + j * _CH, _CH)],
                evbuf.at[buf], sem)
            pltpu.async_copy(dst_hbm.at[s].at[pl.ds(j * _CH, _CH)],
                             idx_v.at[buf], sem)

        def wait_load(j, buf):
            pltpu.make_async_copy(
                src_hbm.at[pl.ds(base + j * _CH, _CH)],
                evbuf.at[buf], sem).wait()
            pltpu.make_async_copy(dst_hbm.at[s].at[pl.ds(j * _CH, _CH)],
                                  idx_v.at[buf], sem).wait()

        def wait_scatter(buf):
            pltpu.make_async_copy(evbuf.at[buf], acc.at[idx_v.at[buf]],
                                  sem2).wait()

        load(0, 0)
        load(1, 1)

        def chunk(j, carry):
            buf = lax.rem(j, _NBUF)
            wait_load(j, buf)
            pltpu.async_copy(evbuf.at[buf], acc.at[idx_v.at[buf]],
                             sem2, add=True)

            @pl.when(j >= 1)
            def _():
                wait_scatter(lax.rem(j + 2, _NBUF))

            @pl.when(j + 2 < _NFULL)
            def _():
                load(j + 2, lax.rem(j + 2, _NBUF))

            return carry

        lax.fori_loop(0, _NFULL, chunk, 0)
        wait_scatter((_NFULL - 1) % _NBUF)
        pltpu.sync_copy(z128.at[pl.ds(0, _CH - _TAIL)],
                        evbuf.at[0].at[pl.ds(_TAIL, _CH - _TAIL)])
        pltpu.sync_copy(src_hbm.at[pl.ds(base + _NFULL * _CH, _TAIL)],
                        evbuf.at[0].at[pl.ds(0, _TAIL)])
        pltpu.sync_copy(dst_hbm.at[s].at[pl.ds(_NFULL * _CH, _CH)],
                        idx_v.at[0])
        pltpu.sync_copy(evbuf.at[0], acc.at[idx_v.at[0]], add=True)

    @pl.when(c == 0)
    def _():
        run(ev_hbm)

    @pl.when(c == 1)
    def _():
        run(exb_hbm)

    plsc.subcore_barrier()

    @pl.when(s < _NS - 1)
    def _():
        pltpu.sync_copy(acc.at[pl.ds(s * _RPS, _RPS)],
                        acc_out.at[c].at[pl.ds(s * _RPS, _RPS)])

    @pl.when(s == _NS - 1)
    def _():
        pltpu.sync_copy(acc.at[pl.ds(s * _RPS, _N - (_NS - 1) * _RPS)],
                        acc_out.at[c].at[pl.ds(s * _RPS, _N - (_NS - 1) * _RPS)])


def kernel(keys, queries, values, edge_index):
    f32 = jnp.float32
    be = 2000
    ev, exb = pl.pallas_call(
        _edge_body,
        grid=(_E // be,),
        in_specs=[pl.BlockSpec((be, _H), lambda i: (i, 0))] * 3,
        out_specs=[pl.BlockSpec((be, _H), lambda i: (i, 0)),
                   pl.BlockSpec((be, _H), lambda i: (i, 0))],
        out_shape=[jax.ShapeDtypeStruct((_E, _H), f32),
                   jax.ShapeDtypeStruct((_E, _H), f32)],
    )(keys, queries, values)

    pad = _NCHK * _CH - _EPS
    dst3 = jnp.pad(edge_index[1].reshape(_NS, _EPS), ((0, 0), (0, pad)))
    z128 = jnp.zeros((_N, _H), f32)

    mesh = plsc.VectorSubcoreMesh(core_axis_name="c", subcore_axis_name="s")
    parts = pl.kernel(
        _scatter_body,
        out_type=jax.ShapeDtypeStruct((_NC, _N, _H), f32),
        mesh=mesh,
        scratch_types=[
            pltpu.VMEM((_NBUF, _CH), jnp.int32),
            pltpu.VMEM((_NBUF, _CH, _H), f32),
            pltpu.VMEM_SHARED((_N, _H), f32),
            pltpu.SemaphoreType.DMA,
            pltpu.SemaphoreType.DMA,
        ],
    )(ev, exb, dst3, z128)

    bn = 2000
    out = pl.pallas_call(
        _combine_body,
        grid=(_N // bn,),
        in_specs=[pl.BlockSpec((1, bn, _H), lambda i: (0, i, 0)),
                  pl.BlockSpec((1, bn, _H), lambda i: (1, i, 0))],
        out_specs=pl.BlockSpec((bn, _H), lambda i: (i, 0)),
        out_shape=jax.ShapeDtypeStruct((_N, _H), f32),
    )(parts, parts)
    return out
